# baseline (speedup 1.0000x reference)
import functools

import jax
import jax.numpy as jnp
from jax import lax
from jax.experimental import pallas as pl
from jax.experimental.pallas import tpu as pltpu
from jax.experimental.pallas import tpu_sc as plsc

N = 10000
D = 128
DW = D // 2
E = 320000
NC, NS, L = 2, 16, 16
C = 125
EPT = E // NS
NCH = EPT // C
RPT = N // NS
ZR = 125
NBUF = 5
NGRP = NCH // NBUF


def _sc_segment_sum(xs, er2, er):
    mesh = plsc.VectorSubcoreMesh(core_axis_name="c", subcore_axis_name="s")

    @functools.partial(
        pl.kernel,
        out_type=jax.ShapeDtypeStruct((NC, N, DW), jnp.float32),
        mesh=mesh,
        scratch_types=[
            pltpu.VMEM((NCH, C), jnp.int32),
            pltpu.VMEM((NCH, C), jnp.int32),
            pltpu.VMEM((NBUF, C, DW), jnp.float32),
            pltpu.VMEM((ZR, DW), jnp.float32),
            pltpu.VMEM_SHARED((N, DW), jnp.float32),
            pltpu.SemaphoreType.DMA((NBUF,)),
            pltpu.SemaphoreType.DMA((NBUF,)),
            pltpu.SemaphoreType.DMA,
        ],
        compiler_params=pltpu.CompilerParams(use_tc_tiling_on_sc=False),
    )
    def seg_kernel(xs_hbm, er2_hbm, er_hbm, out_hbm,
                   src_v, dst_v, gbuf, zbuf, acc_sh, gsem, ssem, zsem):
        cid = lax.axis_index("c")
        sid = lax.axis_index("s")

        cp_src = pltpu.async_copy(er2_hbm.at[cid, sid], src_v, gsem.at[0])
        cp_dst = pltpu.async_copy(er_hbm.at[1, sid], dst_v, gsem.at[1])

        zv = jnp.zeros((L,), jnp.float32)

        @pl.loop(0, ZR)
        def _(r):
            @pl.loop(0, DW, step=L)
            def _(cc):
                zbuf[r, pl.ds(cc, L)] = zv

        base = sid * RPT

        @pl.loop(0, RPT, step=ZR)
        def _(o):
            pltpu.async_copy(zbuf, acc_sh.at[pl.ds(base + o, ZR)], zsem)

        @pl.loop(0, RPT, step=ZR)
        def _(o):
            pltpu.make_async_copy(zbuf, acc_sh.at[pl.ds(base, ZR)], zsem).wait()

        cp_src.wait()
        cp_dst.wait()
        plsc.subcore_barrier()

        def ring(xcol):
            for bb in range(NBUF):
                pltpu.async_copy(xcol.at[src_v.at[bb]], gbuf.at[bb],
                                 gsem.at[bb])

            @pl.loop(0, NGRP)
            def _(g):
                c0 = g * NBUF
                for bb in range(NBUF):
                    c = c0 + bb
                    pltpu.make_async_copy(xcol.at[src_v.at[c]], gbuf.at[bb],
                                          gsem.at[bb]).wait()
                    pltpu.async_copy(gbuf.at[bb], acc_sh.at[dst_v.at[c]],
                                     ssem.at[bb], add=True)
                for bb in range(NBUF):
                    c = c0 + bb
                    pltpu.make_async_copy(gbuf.at[bb], acc_sh.at[dst_v.at[c]],
                                          ssem.at[bb]).wait()

                    @pl.when(c + NBUF < NCH)
                    def _():
                        pltpu.async_copy(xcol.at[src_v.at[c + NBUF]],
                                         gbuf.at[bb], gsem.at[bb])

        ring(xs_hbm)

        plsc.subcore_barrier()

        pltpu.sync_copy(acc_sh.at[pl.ds(base, RPT)],
                        out_hbm.at[cid, pl.ds(base, RPT)])

    return seg_kernel(xs, er2, er)


def _tc_self(x, W_self, b2):
    R = 1000
    G = N // R

    def self_kernel(x_ref, ws_ref, b_ref, o_ref):
        o_ref[...] = jnp.dot(x_ref[...], ws_ref[...],
                             preferred_element_type=jnp.float32) + b_ref[...]

    return pl.pallas_call(
        self_kernel,
        grid=(G,),
        in_specs=[
            pl.BlockSpec((R, D), lambda i: (i, 0)),
            pl.BlockSpec((D, D), lambda i: (0, 0)),
            pl.BlockSpec((1, D), lambda i: (0, 0)),
        ],
        out_specs=pl.BlockSpec((R, D), lambda i: (i, 0)),
        out_shape=jax.ShapeDtypeStruct((N, D), jnp.float32),
    )(x, W_self, b2)


def _tc_head(ha, parts, W_nbr, w2):
    R = 1000
    G = N // R

    def head_kernel(ha_ref, p_ref, wn_ref, w_ref, o_ref):
        agg = jnp.concatenate([p_ref[0], p_ref[1]], axis=-1)
        h = ha_ref[...] + jnp.dot(agg, wn_ref[...],
                                  preferred_element_type=jnp.float32)
        h = jnp.maximum(h, 0.0)
        o_ref[...] = jnp.sum(h * w_ref[...], axis=1, keepdims=True)

    return pl.pallas_call(
        head_kernel,
        grid=(G,),
        in_specs=[
            pl.BlockSpec((R, D), lambda i: (i, 0)),
            pl.BlockSpec((NC, R, DW), lambda i: (0, i, 0)),
            pl.BlockSpec((D, D), lambda i: (0, 0)),
            pl.BlockSpec((1, D), lambda i: (0, 0)),
        ],
        out_specs=pl.BlockSpec((R, 1), lambda i: (i, 0)),
        out_shape=jax.ShapeDtypeStruct((N, 1), jnp.float32),
    )(ha, parts, W_nbr, w2)


@jax.jit
def kernel(x, edge_index, W_self, W_nbr, b, w_out):
    er = edge_index.reshape(2, NS, NCH, C)
    src2 = er[0] * 2
    er2 = jnp.stack([src2, src2 + 1])
    parts = _sc_segment_sum(x.reshape(2 * N, DW), er2, er)
    ha = _tc_self(x, W_self, b.reshape(1, D))
    out = _tc_head(ha, parts, W_nbr, w_out.reshape(1, D))
    return out[:, 0]


# speedup vs baseline: 1.0324x; 1.0324x over previous
import functools

import jax
import jax.numpy as jnp
from jax import lax
from jax.experimental import pallas as pl
from jax.experimental.pallas import tpu as pltpu
from jax.experimental.pallas import tpu_sc as plsc

N = 10000
D = 128
DW = D // 2
E = 320000
NC, NS, L = 2, 16, 16
C = 125
EPT = E // NS
NCH = EPT // C
RPT = N // NS
ZR = 125
NBUF = 5
NGRP = NCH // NBUF


def _sc_segment_sum(xa, xb, er):
    mesh = plsc.VectorSubcoreMesh(core_axis_name="c", subcore_axis_name="s")

    @functools.partial(
        pl.kernel,
        out_type=jax.ShapeDtypeStruct((NC, N, DW), jnp.float32),
        mesh=mesh,
        scratch_types=[
            pltpu.VMEM((NCH, C), jnp.int32),
            pltpu.VMEM((NCH, C), jnp.int32),
            pltpu.VMEM((NBUF, C, DW), jnp.float32),
            pltpu.VMEM((ZR, DW), jnp.float32),
            pltpu.VMEM_SHARED((N, DW), jnp.float32),
            pltpu.SemaphoreType.DMA((NBUF,)),
            pltpu.SemaphoreType.DMA((NBUF,)),
            pltpu.SemaphoreType.DMA,
        ],
        compiler_params=pltpu.CompilerParams(use_tc_tiling_on_sc=False),
    )
    def seg_kernel(xa_hbm, xb_hbm, er_hbm, out_hbm,
                   src_v, dst_v, gbuf, zbuf, acc_sh, gsem, ssem, zsem):
        cid = lax.axis_index("c")
        sid = lax.axis_index("s")

        cp_src = pltpu.async_copy(er_hbm.at[0, sid], src_v, gsem.at[0])
        cp_dst = pltpu.async_copy(er_hbm.at[1, sid], dst_v, gsem.at[1])

        zv = jnp.zeros((L,), jnp.float32)

        @pl.loop(0, ZR)
        def _(r):
            @pl.loop(0, DW, step=L)
            def _(cc):
                zbuf[r, pl.ds(cc, L)] = zv

        base = sid * RPT

        @pl.loop(0, RPT, step=ZR)
        def _(o):
            pltpu.async_copy(zbuf, acc_sh.at[pl.ds(base + o, ZR)], zsem)

        @pl.loop(0, RPT, step=ZR)
        def _(o):
            pltpu.make_async_copy(zbuf, acc_sh.at[pl.ds(base, ZR)], zsem).wait()

        cp_src.wait()
        cp_dst.wait()
        plsc.subcore_barrier()

        def ring(xcol):
            for bb in range(NBUF):
                pltpu.async_copy(xcol.at[src_v.at[bb]], gbuf.at[bb],
                                 gsem.at[bb])

            @pl.loop(0, NGRP)
            def _(g):
                c0 = g * NBUF
                for bb in range(NBUF):
                    c = c0 + bb
                    pltpu.make_async_copy(xcol.at[src_v.at[c]], gbuf.at[bb],
                                          gsem.at[bb]).wait()
                    pltpu.async_copy(gbuf.at[bb], acc_sh.at[dst_v.at[c]],
                                     ssem.at[bb], add=True)
                for bb in range(NBUF):
                    c = c0 + bb
                    pltpu.make_async_copy(gbuf.at[bb], acc_sh.at[dst_v.at[c]],
                                          ssem.at[bb]).wait()

                    @pl.when(c + NBUF < NCH)
                    def _():
                        pltpu.async_copy(xcol.at[src_v.at[c + NBUF]],
                                         gbuf.at[bb], gsem.at[bb])

        @pl.when(cid == 0)
        def _():
            ring(xa_hbm)

        @pl.when(cid == 1)
        def _():
            ring(xb_hbm)

        plsc.subcore_barrier()

        pltpu.sync_copy(acc_sh.at[pl.ds(base, RPT)],
                        out_hbm.at[cid, pl.ds(base, RPT)])

    return seg_kernel(xa, xb, er)


def _tc_self(x, W_self, b2):
    R = 1000
    G = N // R

    def self_kernel(x_ref, ws_ref, b_ref, o_ref):
        o_ref[...] = jnp.dot(x_ref[...], ws_ref[...],
                             preferred_element_type=jnp.float32) + b_ref[...]

    return pl.pallas_call(
        self_kernel,
        grid=(G,),
        in_specs=[
            pl.BlockSpec((R, D), lambda i: (i, 0)),
            pl.BlockSpec((D, D), lambda i: (0, 0)),
            pl.BlockSpec((1, D), lambda i: (0, 0)),
        ],
        out_specs=pl.BlockSpec((R, D), lambda i: (i, 0)),
        out_shape=jax.ShapeDtypeStruct((N, D), jnp.float32),
    )(x, W_self, b2)


def _tc_head(ha, parts, W_nbr, w2):
    R = 1000
    G = N // R

    def head_kernel(ha_ref, p_ref, wn_ref, w_ref, o_ref):
        agg = jnp.concatenate([p_ref[0], p_ref[1]], axis=-1)
        h = ha_ref[...] + jnp.dot(agg, wn_ref[...],
                                  preferred_element_type=jnp.float32)
        h = jnp.maximum(h, 0.0)
        o_ref[...] = jnp.sum(h * w_ref[...], axis=1, keepdims=True)

    return pl.pallas_call(
        head_kernel,
        grid=(G,),
        in_specs=[
            pl.BlockSpec((R, D), lambda i: (i, 0)),
            pl.BlockSpec((NC, R, DW), lambda i: (0, i, 0)),
            pl.BlockSpec((D, D), lambda i: (0, 0)),
            pl.BlockSpec((1, D), lambda i: (0, 0)),
        ],
        out_specs=pl.BlockSpec((R, 1), lambda i: (i, 0)),
        out_shape=jax.ShapeDtypeStruct((N, 1), jnp.float32),
    )(ha, parts, W_nbr, w2)


@jax.jit
def kernel(x, edge_index, W_self, W_nbr, b, w_out):
    er = edge_index.reshape(2, NS, NCH, C)
    parts = _sc_segment_sum(x[:, :DW], x[:, DW:], er)
    ha = _tc_self(x, W_self, b.reshape(1, D))
    out = _tc_head(ha, parts, W_nbr, w_out.reshape(1, D))
    return out[:, 0]
